# Initial kernel scaffold; baseline (speedup 1.0000x reference)
import jax
import jax.numpy as jnp
from jax.experimental import pallas as pl


def kernel(images, feat0, feat1, feat2, feat3, feat4, conv_w, conv_b, cls_w, cls_b, bbox_w, bbox_b):
    raise NotImplementedError("write your pallas kernel here")


# speedup vs baseline: 1.4650x; 1.4650x over previous
import functools
import math

import jax
import jax.numpy as jnp
from jax.experimental import pallas as pl
from jax.experimental.pallas import tpu as pltpu

_SIZES = [32.0, 64.0, 128.0, 256.0, 512.0]
_RATIOS = [0.5, 1.0, 2.0]
_NUM_COMBOS = len(_SIZES) * len(_RATIOS)
_C = 256
_NHEAD = 75
_PAD = 128


def _dot(a, b):
    return jax.lax.dot_general(a, b, (((1,), (0,)), ((), ())),
                               preferred_element_type=jnp.float32)


def _anchors(a_ref, whc_ref, g, tn, stride, p):
    lane = jax.lax.broadcasted_iota(jnp.int32, (1, tn), 1)
    n = lane + p * tn
    i = n // g
    xv = i.astype(jnp.float32) * stride
    yv = (n - i * g).astype(jnp.float32) * stride
    rr = jax.lax.broadcasted_iota(jnp.int32, (4 * _NUM_COMBOS, 1), 0)
    f = rr % 4
    a_ref[...] = jnp.where(f == 0, xv,
                           jnp.where(f == 1, yv, whc_ref[:, 0:1]))


def _finish(t_acc, cb_ref, wh_ref, hb_ref, o_ref):
    t = jnp.maximum(t_acc + cb_ref[:, 0:1], 0.0)
    o_ref[...] = _dot(wh_ref[...], t) + hb_ref[:, 0:1]


def _body_l0(xp_ref, w9_ref, cb_ref, wh_ref, hb_ref, whc_ref, o_ref, a_ref, *,
             g, tn, stride):
    p = pl.program_id(0)
    base = _PAD + p * tn
    wins = [xp_ref[:, pl.ds(base + (dy - 1) * g, tn)] for dy in range(3)]
    sums = []
    for dx in range(3):
        s = _dot(w9_ref[dx], wins[0])
        for dy in (1, 2):
            s += _dot(w9_ref[dy * 3 + dx], wins[dy])
        sums.append(s)
    lane = jax.lax.broadcasted_iota(jnp.int32, (1, tn), 1)
    j = lane % g
    acc = sums[1]
    acc += jnp.where(j != 0, pltpu.roll(sums[0], 1, axis=1), 0.0)
    acc += jnp.where(j != g - 1, pltpu.roll(sums[2], tn - 1, axis=1), 0.0)
    _finish(acc, cb_ref, wh_ref, hb_ref, o_ref)
    _anchors(a_ref, whc_ref, g, tn, stride, p)


def _body_tail(x_ref, w9_ref, cb_ref, wh_ref, hb_ref, whc_ref, o_ref, a_ref, *,
               g, tn, stride):
    win = x_ref[...]
    n = jax.lax.broadcasted_iota(jnp.int32, (1, tn), 1)
    j = n % g
    mv = [n >= g, None, n < tn - g]
    mh = [j != 0, None, j != g - 1]
    acc = jnp.zeros((_C, tn), jnp.float32)
    for dy in range(3):
        for dx in range(3):
            r = _dot(w9_ref[dy * 3 + dx], win)
            off = (dy - 1) * g + (dx - 1)
            if off:
                r = pltpu.roll(r, (-off) % tn, axis=1)
            m = mv[dy] if mh[dx] is None else (
                mh[dx] if mv[dy] is None else mv[dy] & mh[dx])
            acc += r if m is None else jnp.where(m, r, 0.0)
    _finish(acc, cb_ref, wh_ref, hb_ref, o_ref)
    _anchors(a_ref, whc_ref, g, tn, stride, 0)


def _level(feat, w9, cb, wh, hb, whc, img_size):
    g = feat.shape[-1]
    n = g * g
    stride = float(img_size) / g
    x = feat.reshape(_C, n)
    tiled = n > 4096
    if tiled:
        tn, xin = 2048, jnp.pad(x, ((0, 0), (_PAD, _PAD)))
        body = functools.partial(_body_l0, g=g, tn=tn, stride=stride)
    else:
        tn, xin = n, x
        body = functools.partial(_body_tail, g=g, tn=tn, stride=stride)
    grid = n // tn
    out, anch = pl.pallas_call(
        body,
        grid=(grid,),
        in_specs=[
            pl.BlockSpec(xin.shape, lambda p: (0, 0)),
            pl.BlockSpec(w9.shape, lambda p: (0, 0, 0)),
            pl.BlockSpec(cb.shape, lambda p: (0, 0)),
            pl.BlockSpec(wh.shape, lambda p: (0, 0)),
            pl.BlockSpec(hb.shape, lambda p: (0, 0)),
            pl.BlockSpec(whc.shape, lambda p: (0, 0)),
        ],
        out_specs=[
            pl.BlockSpec((_NHEAD, tn), lambda p: (0, p)),
            pl.BlockSpec((4 * _NUM_COMBOS, tn), lambda p: (0, p)),
        ],
        out_shape=[
            jax.ShapeDtypeStruct((_NHEAD, n), jnp.float32),
            jax.ShapeDtypeStruct((4 * _NUM_COMBOS, n), jnp.float32),
        ],
    )(xin, w9, cb, wh, hb, whc)
    logits = out[:15].reshape(1, 15, g, g)
    bbox = out[15:].reshape(1, 60, g, g)
    anchors = anch.reshape(_NUM_COMBOS, 4, n).transpose(0, 2, 1).reshape(-1, 4)
    return logits, bbox, anchors


def kernel(images, feat0, feat1, feat2, feat3, feat4,
           conv_w, conv_b, cls_w, cls_b, bbox_w, bbox_b):
    img_size = images.shape[-1]
    w9 = conv_w.transpose(2, 3, 0, 1).reshape(9, _C, _C)
    cb = conv_b.reshape(_C, 1)
    wh = jnp.concatenate([cls_w, bbox_w], axis=0)
    hb = jnp.concatenate([cls_b, bbox_b]).reshape(_NHEAD, 1)
    whc = []
    for s in _SIZES:
        for r in _RATIOS:
            rs = math.sqrt(r)
            whc += [0.0, 0.0, s * rs, s / rs]
    whc = jnp.asarray(whc, jnp.float32).reshape(4 * _NUM_COMBOS, 1)
    logits, bbox, anchors = [], [], []
    for feat in (feat0, feat1, feat2, feat3, feat4):
        l, b, a = _level(feat, w9, cb, wh, hb, whc, img_size)
        logits.append(l)
        bbox.append(b)
        anchors.append(a)
    return (jnp.concatenate(anchors, axis=0), *logits, *bbox)


# speedup vs baseline: 1.4855x; 1.0140x over previous
import functools
import math

import jax
import jax.numpy as jnp
from jax.experimental import pallas as pl
from jax.experimental.pallas import tpu as pltpu

_SIZES = [32.0, 64.0, 128.0, 256.0, 512.0]
_RATIOS = [0.5, 1.0, 2.0]
_NUM_COMBOS = len(_SIZES) * len(_RATIOS)
_C = 256
_NHEAD = 75
_PAD = 128


def _dot(a, b):
    return jax.lax.dot_general(a, b, (((1,), (0,)), ((), ())),
                               preferred_element_type=jnp.float32)


def _anchors(a_ref, whc_ref, g, tn, stride, p):
    lane = jax.lax.broadcasted_iota(jnp.int32, (1, tn), 1)
    n = lane + p * tn
    i = n // g
    xv = i.astype(jnp.float32) * stride
    yv = (n - i * g).astype(jnp.float32) * stride
    rr = jax.lax.broadcasted_iota(jnp.int32, (4 * _NUM_COMBOS, 1), 0)
    f = rr % 4
    a_ref[...] = jnp.where(f == 0, xv,
                           jnp.where(f == 1, yv, whc_ref[:, 0:1]))


def _finish(t_acc, cb_ref, wh_ref, hb_ref, o_ref):
    t = jnp.maximum(t_acc + cb_ref[:, 0:1], 0.0)
    o_ref[...] = _dot(wh_ref[...], t.astype(jnp.bfloat16)) + hb_ref[:, 0:1]


def _body_l0(xp_ref, w9_ref, cb_ref, wh_ref, hb_ref, whc_ref, o_ref, a_ref, *,
             g, tn, stride):
    p = pl.program_id(0)
    base = _PAD + p * tn
    wins = [xp_ref[:, pl.ds(base + (dy - 1) * g, tn)] for dy in range(3)]
    sums = []
    for dx in range(3):
        s = _dot(w9_ref[dx], wins[0])
        for dy in (1, 2):
            s += _dot(w9_ref[dy * 3 + dx], wins[dy])
        sums.append(s)
    lane = jax.lax.broadcasted_iota(jnp.int32, (1, tn), 1)
    j = lane % g
    acc = sums[1]
    acc += jnp.where(j != 0, pltpu.roll(sums[0], 1, axis=1), 0.0)
    acc += jnp.where(j != g - 1, pltpu.roll(sums[2], tn - 1, axis=1), 0.0)
    _finish(acc, cb_ref, wh_ref, hb_ref, o_ref)
    _anchors(a_ref, whc_ref, g, tn, stride, p)


def _body_tail(x_ref, w9_ref, cb_ref, wh_ref, hb_ref, whc_ref, o_ref, a_ref, *,
               g, tn, stride):
    win = x_ref[...]
    n = jax.lax.broadcasted_iota(jnp.int32, (1, tn), 1)
    j = n % g
    mv = [n >= g, None, n < tn - g]
    mh = [j != 0, None, j != g - 1]
    acc = jnp.zeros((_C, tn), jnp.float32)
    for dy in range(3):
        for dx in range(3):
            r = _dot(w9_ref[dy * 3 + dx], win)
            off = (dy - 1) * g + (dx - 1)
            if off:
                r = pltpu.roll(r, (-off) % tn, axis=1)
            m = mv[dy] if mh[dx] is None else (
                mh[dx] if mv[dy] is None else mv[dy] & mh[dx])
            acc += r if m is None else jnp.where(m, r, 0.0)
    _finish(acc, cb_ref, wh_ref, hb_ref, o_ref)
    _anchors(a_ref, whc_ref, g, tn, stride, 0)


def _level(feat, w9, cb, wh, hb, whc, img_size):
    g = feat.shape[-1]
    n = g * g
    stride = float(img_size) / g
    x = feat.reshape(_C, n).astype(jnp.bfloat16)
    tiled = n > 4096
    if tiled:
        tn, xin = 2048, jnp.pad(x, ((0, 0), (_PAD, _PAD)))
        body = functools.partial(_body_l0, g=g, tn=tn, stride=stride)
    else:
        tn, xin = n, x
        body = functools.partial(_body_tail, g=g, tn=tn, stride=stride)
    grid = n // tn
    out, anch = pl.pallas_call(
        body,
        grid=(grid,),
        in_specs=[
            pl.BlockSpec(xin.shape, lambda p: (0, 0)),
            pl.BlockSpec(w9.shape, lambda p: (0, 0, 0)),
            pl.BlockSpec(cb.shape, lambda p: (0, 0)),
            pl.BlockSpec(wh.shape, lambda p: (0, 0)),
            pl.BlockSpec(hb.shape, lambda p: (0, 0)),
            pl.BlockSpec(whc.shape, lambda p: (0, 0)),
        ],
        out_specs=[
            pl.BlockSpec((_NHEAD, tn), lambda p: (0, p)),
            pl.BlockSpec((4 * _NUM_COMBOS, tn), lambda p: (0, p)),
        ],
        out_shape=[
            jax.ShapeDtypeStruct((_NHEAD, n), jnp.float32),
            jax.ShapeDtypeStruct((4 * _NUM_COMBOS, n), jnp.float32),
        ],
    )(xin, w9, cb, wh, hb, whc)
    logits = out[:15].reshape(1, 15, g, g)
    bbox = out[15:].reshape(1, 60, g, g)
    anchors = anch.reshape(_NUM_COMBOS, 4, n).transpose(0, 2, 1).reshape(-1, 4)
    return logits, bbox, anchors


def kernel(images, feat0, feat1, feat2, feat3, feat4,
           conv_w, conv_b, cls_w, cls_b, bbox_w, bbox_b):
    img_size = images.shape[-1]
    w9 = conv_w.transpose(2, 3, 0, 1).reshape(9, _C, _C).astype(jnp.bfloat16)
    cb = conv_b.reshape(_C, 1)
    wh = jnp.concatenate([cls_w, bbox_w], axis=0).astype(jnp.bfloat16)
    hb = jnp.concatenate([cls_b, bbox_b]).reshape(_NHEAD, 1)
    whc = []
    for s in _SIZES:
        for r in _RATIOS:
            rs = math.sqrt(r)
            whc += [0.0, 0.0, s * rs, s / rs]
    whc = jnp.asarray(whc, jnp.float32).reshape(4 * _NUM_COMBOS, 1)
    logits, bbox, anchors = [], [], []
    for feat in (feat0, feat1, feat2, feat3, feat4):
        l, b, a = _level(feat, w9, cb, wh, hb, whc, img_size)
        logits.append(l)
        bbox.append(b)
        anchors.append(a)
    return (jnp.concatenate(anchors, axis=0), *logits, *bbox)
